# baseline (speedup 1.0000x reference)
import jax
import jax.numpy as jnp
from jax.experimental import pallas as pl

_N_PTS = 128
_MIN_DEPTH = 0.1
_MAX_DEPTH = 10.0
_BL = 2048


def _body(R_ref, T_ref, xy_ref, o_ref, d_ref, l_ref, xys_ref):
    x = xy_ref[0, :, 0:1]
    y = xy_ref[0, :, 1:2]
    xs = x * 2.0 - 1.0
    ys = y * 2.0 - 1.0

    r00 = R_ref[0, 0, 0]
    r01 = R_ref[0, 0, 1]
    r02 = R_ref[0, 0, 2]
    r10 = R_ref[0, 1, 0]
    r11 = R_ref[0, 1, 1]
    r12 = R_ref[0, 1, 2]
    r20 = R_ref[0, 2, 0]
    r21 = R_ref[0, 2, 1]
    r22 = R_ref[0, 2, 2]
    t0 = T_ref[0, 0, 0]
    t1 = T_ref[0, 0, 1]
    t2 = T_ref[0, 0, 2]

    d0 = xs * r00 + ys * r01 + r02
    d1 = xs * r10 + ys * r11 + r12
    d2 = xs * r20 + ys * r21 + r22
    norm = jnp.sqrt(d0 * d0 + d1 * d1 + d2 * d2)
    inv = 1.0 / jnp.maximum(norm, 1e-12)
    d0 = d0 * inv
    d1 = d1 * inv
    d2 = d2 * inv

    c0 = -(t0 * r00 + t1 * r01 + t2 * r02)
    c1 = -(t0 * r10 + t1 * r11 + t2 * r12)
    c2 = -(t0 * r20 + t1 * r21 + t2 * r22)

    xys_ref[0, :, 0:1] = xs
    xys_ref[0, :, 1:2] = ys
    d_ref[0] = jnp.concatenate([d0, d1, d2], axis=1)
    ones = jnp.ones((_BL, 1), jnp.float32)
    o_ref[0] = jnp.concatenate([ones * c0, ones * c1, ones * c2], axis=1)

    step = (_MAX_DEPTH - _MIN_DEPTH) / (_N_PTS - 1)
    iota = jax.lax.broadcasted_iota(jnp.int32, (_BL, _N_PTS), 1).astype(jnp.float32)
    l_ref[0] = _MIN_DEPTH + iota * step


def kernel(R, T, xy):
    B, n_rays, _ = xy.shape
    nb = n_rays // _BL
    T3 = T.reshape(B, 1, 3)
    out = pl.pallas_call(
        _body,
        grid=(B, nb),
        in_specs=[
            pl.BlockSpec((1, 3, 3), lambda b, i: (b, 0, 0)),
            pl.BlockSpec((1, 1, 3), lambda b, i: (b, 0, 0)),
            pl.BlockSpec((1, _BL, 2), lambda b, i: (b, i, 0)),
        ],
        out_specs=[
            pl.BlockSpec((1, _BL, 3), lambda b, i: (b, i, 0)),
            pl.BlockSpec((1, _BL, 3), lambda b, i: (b, i, 0)),
            pl.BlockSpec((1, _BL, _N_PTS), lambda b, i: (b, i, 0)),
            pl.BlockSpec((1, _BL, 2), lambda b, i: (b, i, 0)),
        ],
        out_shape=[
            jax.ShapeDtypeStruct((B, n_rays, 3), jnp.float32),
            jax.ShapeDtypeStruct((B, n_rays, 3), jnp.float32),
            jax.ShapeDtypeStruct((B, n_rays, _N_PTS), jnp.float32),
            jax.ShapeDtypeStruct((B, n_rays, 2), jnp.float32),
        ],
    )(R, T3, xy)
    origins, directions, lengths, xys = out
    return (origins, directions, lengths, xys)


# speedup vs baseline: 10.4074x; 10.4074x over previous
import jax
import jax.numpy as jnp
from jax.experimental import pallas as pl

_NP = 128
_MIN_DEPTH = 0.1
_MAX_DEPTH = 10.0
_RC = 2048
_BG = 8


def _body(feat_ref, x_ref, y_ref, xyp_ref, dir_ref, org_ref, len_ref, xys_ref):
    f = feat_ref[...]
    r00 = f[:, 0:1]
    r01 = f[:, 1:2]
    r02 = f[:, 2:3]
    r10 = f[:, 3:4]
    r11 = f[:, 4:5]
    r12 = f[:, 5:6]
    r20 = f[:, 6:7]
    r21 = f[:, 7:8]
    r22 = f[:, 8:9]
    t0 = f[:, 9:10]
    t1 = f[:, 10:11]
    t2 = f[:, 11:12]

    xys_ref[...] = xyp_ref[...] * 2.0 - 1.0

    xs = x_ref[...] * 2.0 - 1.0
    ys = y_ref[...] * 2.0 - 1.0

    d0 = xs * r00 + ys * r01 + r02
    d1 = xs * r10 + ys * r11 + r12
    d2 = xs * r20 + ys * r21 + r22
    norm = jnp.sqrt(d0 * d0 + d1 * d1 + d2 * d2)
    inv = 1.0 / jnp.maximum(norm, 1e-12)
    dir_ref[0] = d0 * inv
    dir_ref[1] = d1 * inv
    dir_ref[2] = d2 * inv

    c0 = -(t0 * r00 + t1 * r01 + t2 * r02)
    c1 = -(t0 * r10 + t1 * r11 + t2 * r12)
    c2 = -(t0 * r20 + t1 * r21 + t2 * r22)
    org_ref[0] = jnp.broadcast_to(c0, (_BG, _RC))
    org_ref[1] = jnp.broadcast_to(c1, (_BG, _RC))
    org_ref[2] = jnp.broadcast_to(c2, (_BG, _RC))

    step = (_MAX_DEPTH - _MIN_DEPTH) / (_NP - 1)
    iota = jax.lax.broadcasted_iota(jnp.int32, (1, 1, _NP), 2).astype(jnp.float32)
    len_ref[...] = jnp.broadcast_to(_MIN_DEPTH + iota * step, (_BG, _RC, _NP))


def kernel(R, T, xy):
    B, n, _ = xy.shape
    feat = jnp.concatenate([R.reshape(B, 9), T], axis=1)
    x2d = xy[:, :, 0]
    y2d = xy[:, :, 1]
    xyp = xy.reshape(B, n // 128, 128, 2).transpose(0, 1, 3, 2).reshape(B, n // 64, 128)

    ng = B // _BG
    nc = n // _RC
    rows = _RC // 64

    dir_p, org_p, lengths, xys_p = pl.pallas_call(
        _body,
        grid=(ng, nc),
        in_specs=[
            pl.BlockSpec((_BG, 12), lambda g, i: (g, 0)),
            pl.BlockSpec((_BG, _RC), lambda g, i: (g, i)),
            pl.BlockSpec((_BG, _RC), lambda g, i: (g, i)),
            pl.BlockSpec((_BG, rows, 128), lambda g, i: (g, i, 0)),
        ],
        out_specs=[
            pl.BlockSpec((3, _BG, _RC), lambda g, i: (0, g, i)),
            pl.BlockSpec((3, _BG, _RC), lambda g, i: (0, g, i)),
            pl.BlockSpec((_BG, _RC, _NP), lambda g, i: (g, i, 0)),
            pl.BlockSpec((_BG, rows, 128), lambda g, i: (g, i, 0)),
        ],
        out_shape=[
            jax.ShapeDtypeStruct((3, B, n), jnp.float32),
            jax.ShapeDtypeStruct((3, B, n), jnp.float32),
            jax.ShapeDtypeStruct((B, n, _NP), jnp.float32),
            jax.ShapeDtypeStruct((B, n // 64, 128), jnp.float32),
        ],
    )(feat, x2d, y2d, xyp)

    directions = dir_p.transpose(1, 2, 0)
    origins = org_p.transpose(1, 2, 0)
    xys = xys_p.reshape(B, n // 128, 2, 128).transpose(0, 1, 3, 2).reshape(B, n, 2)
    return (origins, directions, lengths, xys)


# speedup vs baseline: 11.7208x; 1.1262x over previous
import jax
import jax.numpy as jnp
from jax.experimental import pallas as pl

_NP = 128
_MIN_DEPTH = 0.1
_MAX_DEPTH = 10.0
_RC = 2048
_BG = 8


def _body(feat_ref, xyp_ref, dir_ref, org_ref, len_ref, xys_ref):
    f = feat_ref[...]
    r00 = f[:, 0:1]
    r01 = f[:, 1:2]
    r02 = f[:, 2:3]
    r10 = f[:, 3:4]
    r11 = f[:, 4:5]
    r12 = f[:, 5:6]
    r20 = f[:, 6:7]
    r21 = f[:, 7:8]
    r22 = f[:, 8:9]
    t0 = f[:, 9:10]
    t1 = f[:, 10:11]
    t2 = f[:, 11:12]

    a = xyp_ref[...] * 2.0 - 1.0
    xys_ref[...] = a

    for q in range(_RC // 128):
        xs = a[:, 2 * q, :]
        ys = a[:, 2 * q + 1, :]
        d0 = xs * r00 + ys * r01 + r02
        d1 = xs * r10 + ys * r11 + r12
        d2 = xs * r20 + ys * r21 + r22
        norm = jnp.sqrt(d0 * d0 + d1 * d1 + d2 * d2)
        inv = 1.0 / jnp.maximum(norm, 1e-12)
        sl = pl.ds(q * 128, 128)
        dir_ref[0, :, sl] = d0 * inv
        dir_ref[1, :, sl] = d1 * inv
        dir_ref[2, :, sl] = d2 * inv

    c0 = -(t0 * r00 + t1 * r01 + t2 * r02)
    c1 = -(t0 * r10 + t1 * r11 + t2 * r12)
    c2 = -(t0 * r20 + t1 * r21 + t2 * r22)
    org_ref[0] = jnp.broadcast_to(c0, (_BG, _RC))
    org_ref[1] = jnp.broadcast_to(c1, (_BG, _RC))
    org_ref[2] = jnp.broadcast_to(c2, (_BG, _RC))

    step = (_MAX_DEPTH - _MIN_DEPTH) / (_NP - 1)
    iota = jax.lax.broadcasted_iota(jnp.int32, (1, 1, _NP), 2).astype(jnp.float32)
    len_ref[...] = jnp.broadcast_to(_MIN_DEPTH + iota * step, (_BG, _RC, _NP))


def kernel(R, T, xy):
    B, n, _ = xy.shape
    feat = jnp.concatenate([R.reshape(B, 9), T], axis=1)
    xyp = xy.reshape(B, n // 128, 128, 2).transpose(0, 1, 3, 2).reshape(B, n // 64, 128)

    ng = B // _BG
    nc = n // _RC
    rows = _RC // 64

    dir_p, org_p, lengths, xys_p = pl.pallas_call(
        _body,
        grid=(ng, nc),
        in_specs=[
            pl.BlockSpec((_BG, 12), lambda g, i: (g, 0)),
            pl.BlockSpec((_BG, rows, 128), lambda g, i: (g, i, 0)),
        ],
        out_specs=[
            pl.BlockSpec((3, _BG, _RC), lambda g, i: (0, g, i)),
            pl.BlockSpec((3, _BG, _RC), lambda g, i: (0, g, i)),
            pl.BlockSpec((_BG, _RC, _NP), lambda g, i: (g, i, 0)),
            pl.BlockSpec((_BG, rows, 128), lambda g, i: (g, i, 0)),
        ],
        out_shape=[
            jax.ShapeDtypeStruct((3, B, n), jnp.float32),
            jax.ShapeDtypeStruct((3, B, n), jnp.float32),
            jax.ShapeDtypeStruct((B, n, _NP), jnp.float32),
            jax.ShapeDtypeStruct((B, n // 64, 128), jnp.float32),
        ],
    )(feat, xyp)

    directions = dir_p.transpose(1, 2, 0)
    origins = org_p.transpose(1, 2, 0)
    xys = xys_p.reshape(B, n // 128, 2, 128).transpose(0, 1, 3, 2).reshape(B, n, 2)
    return (origins, directions, lengths, xys)
